# baseline (speedup 1.0000x reference)
import functools

import numpy as np

import jax
import jax.numpy as jnp
from jax import lax
from jax.experimental import pallas as pl
from jax.experimental.pallas import tpu as pltpu
from jax.experimental.pallas import tpu_sc as plsc

N = 10000
E = 320000
D = 128
DH = D // 2
NC = 2
NS = 16
CH = 64
SB = 6
NSUP = 54
NCH = SB * NSUP
EPAD = NS * NCH * CH
NPAD = 10240
RPT = NPAD // NS
DEGW = 8
DNCH = NCH // NC

_mesh = plsc.VectorSubcoreMesh(core_axis_name="c", subcore_axis_name="s")


@functools.partial(
    pl.kernel,
    out_type=jax.ShapeDtypeStruct((NC, NPAD, DEGW), jnp.float32),
    mesh=_mesh,
    scratch_types=[
        pltpu.VMEM((2, CH), jnp.int32),
        pltpu.VMEM((2 * CH, DEGW), jnp.float32),
        pltpu.SemaphoreType.DMA,
        pltpu.SemaphoreType.DMA,
        pltpu.VMEM_SHARED((NPAD, DEGW), jnp.float32),
    ],
)
def _deg_kernel(eidx_hbm, vals_hbm, out_hbm, dbuf, vbuf, sem_a, sem_b, acc):
    c = lax.axis_index("c")
    s = lax.axis_index("s")
    pltpu.sync_copy(vals_hbm, vbuf)
    for k in range(RPT // CH):
        pltpu.sync_copy(vbuf.at[pl.ds(CH, CH)],
                        acc.at[pl.ds(s * RPT + k * CH, CH)])
    plsc.subcore_barrier()

    lo = c * DNCH
    hi = lo + DNCH - 1
    pltpu.make_async_copy(eidx_hbm.at[s, lo, 1], dbuf.at[0], sem_a).start()
    pltpu.make_async_copy(eidx_hbm.at[s, lo + 1, 1], dbuf.at[1], sem_b).start()

    def pair(i, carry):
        j0 = lo + 2 * i
        j1 = j0 + 1
        pltpu.make_async_copy(eidx_hbm.at[s, j0, 1], dbuf.at[0], sem_a).wait()
        pltpu.sync_copy(vbuf.at[pl.ds(0, CH)], acc.at[dbuf.at[0]], add=True)
        pltpu.make_async_copy(eidx_hbm.at[s, jnp.minimum(j0 + 2, hi), 1],
                              dbuf.at[0], sem_a).start()
        pltpu.make_async_copy(eidx_hbm.at[s, j1, 1], dbuf.at[1], sem_b).wait()
        pltpu.sync_copy(vbuf.at[pl.ds(0, CH)], acc.at[dbuf.at[1]], add=True)
        pltpu.make_async_copy(eidx_hbm.at[s, jnp.minimum(j1 + 2, hi), 1],
                              dbuf.at[1], sem_b).start()
        return carry

    lax.fori_loop(0, DNCH // 2, pair, 0)
    pltpu.make_async_copy(eidx_hbm.at[s, hi, 1], dbuf.at[0], sem_a).wait()
    pltpu.make_async_copy(eidx_hbm.at[s, hi, 1], dbuf.at[1], sem_b).wait()
    plsc.subcore_barrier()

    for k in range(RPT // CH):
        stage = vbuf.at[pl.ds(0, CH)]
        pltpu.sync_copy(acc.at[pl.ds(s * RPT + k * CH, CH)], stage)
        pltpu.sync_copy(stage, out_hbm.at[c, pl.ds(s * RPT + k * CH, CH)])


@functools.partial(
    pl.kernel,
    out_type=jax.ShapeDtypeStruct((NC, NPAD, DH), jnp.float32),
    mesh=_mesh,
    scratch_types=[
        pltpu.VMEM((2, SB, 2, CH), jnp.int32),
        pltpu.VMEM((3, CH, DH // 2), jnp.int32),
        pltpu.VMEM((2, CH, DH), jnp.float32),
        pltpu.SemaphoreType.DMA,
        pltpu.SemaphoreType.DMA,
        pltpu.SemaphoreType.DMA,
        pltpu.SemaphoreType.DMA,
        pltpu.SemaphoreType.DMA,
        pltpu.SemaphoreType.DMA,
        pltpu.VMEM_SHARED((NPAD, DH), jnp.float32),
    ],
    compiler_params=pltpu.CompilerParams(use_tc_tiling_on_sc=False),
)
def _agg_kernel(xs_hbm, eidx_hbm, out_hbm, sbuf, rows, rows_s, sem_i,
                g0, g1, g2, s0, s1, acc):
    c = lax.axis_index("c")
    s = lax.axis_index("s")
    xs_c = xs_hbm.at[c]
    gsem = (g0, g1, g2)
    ssem = (s0, s1)

    def g_start(qu, cc, b):
        pltpu.make_async_copy(xs_c.at[sbuf.at[qu, cc, 0]],
                              rows.at[b], gsem[b]).start()

    def g_wait(qu, cc, b):
        pltpu.make_async_copy(xs_c.at[sbuf.at[qu, cc, 0]],
                              rows.at[b], gsem[b]).wait()

    def s_start(qu, cc, b):
        pltpu.make_async_copy(rows_s.at[b], acc.at[sbuf.at[qu, cc, 1]],
                              ssem[b]).start()

    def s_wait(qu, cc, b):
        pltpu.make_async_copy(rows_s.at[b], acc.at[sbuf.at[qu, cc, 1]],
                              ssem[b]).wait()

    def convert(b3, b2):
        def conv_row(r, carry):
            for g in range(DH // 32):
                w = rows[b3, r, pl.ds(g * 16, 16)]
                lo = lax.bitcast_convert_type(w << 16, jnp.float32)
                hi = lax.bitcast_convert_type(w & jnp.int32(-65536), jnp.float32)
                rows_s[b2, r, pl.ds(g * 32, 16)] = lo
                rows_s[b2, r, pl.ds(g * 32 + 16, 16)] = hi
            return carry

        lax.fori_loop(0, CH, conv_row, 0)

    def i_start(v, slot):
        pltpu.make_async_copy(eidx_hbm.at[s, pl.ds(v * SB, SB)],
                              sbuf.at[slot], sem_i).start()

    def i_wait(v, slot):
        pltpu.make_async_copy(eidx_hbm.at[s, pl.ds(v * SB, SB)],
                              sbuf.at[slot], sem_i).wait()

    def fill_zero(i, carry):
        for k in range(DH // 16):
            rows_s[0, i, pl.ds(k * 16, 16)] = jnp.zeros((16,), jnp.float32)
        return carry

    lax.fori_loop(0, CH, fill_zero, 0)
    for k in range(RPT // CH):
        pltpu.sync_copy(rows_s.at[0], acc.at[pl.ds(s * RPT + k * CH, CH)])
    plsc.subcore_barrier()

    pltpu.sync_copy(eidx_hbm.at[s, pl.ds(0, SB)], sbuf.at[0])
    g_start(0, 0, 0)
    g_start(0, 1, 1)
    for kk in range(SB):
        b3 = kk % 3
        b2 = kk % 2
        g_wait(0, kk, b3)
        convert(b3, b2)
        s_start(0, kk, b2)
        if kk == 0:
            i_start(1, 1)
        else:
            s_wait(0, kk - 1, (kk - 1) % 2)
        if kk < SB - 2:
            g_start(0, kk + 2, (kk + 2) % 3)
        else:
            if kk == SB - 2:
                i_wait(1, 1)
            g_start(1, kk + 2 - SB, (kk + 2) % 3)

    def super_body(v, carry):
        pv = lax.rem(v, 2)
        qv = 1 - pv
        not_last = v < NSUP - 1
        for kk in range(SB):
            b3 = kk % 3
            b2 = kk % 2
            g_wait(pv, kk, b3)
            convert(b3, b2)
            s_start(pv, kk, b2)
            if kk == 0:
                s_wait(qv, SB - 1, 1)

                @pl.when(v + 1 < NSUP)
                def _():
                    i_start(v + 1, qv)
            else:
                s_wait(pv, kk - 1, (kk - 1) % 2)
            if kk < SB - 2:
                g_start(pv, kk + 2, (kk + 2) % 3)
            else:
                if kk == SB - 2:
                    @pl.when(not_last)
                    def _():
                        i_wait(v + 1, qv)
                qu = jnp.where(not_last, qv, pv)
                cc = jnp.where(not_last, kk + 2 - SB, SB - 1)
                g_start(qu, cc, (kk + 2) % 3)
        return carry

    lax.fori_loop(1, NSUP, super_body, 0)

    pu_last = (NSUP - 1) % 2
    s_wait(pu_last, SB - 1, (SB - 1) % 2)
    g_wait(pu_last, SB - 1, 0)
    g_wait(pu_last, SB - 1, 1)
    plsc.subcore_barrier()

    for k in range(RPT // CH):
        buf = rows_s.at[k % 2]
        pltpu.sync_copy(acc.at[pl.ds(s * RPT + k * CH, CH)], buf)
        pltpu.sync_copy(buf, out_hbm.at[c, pl.ds(s * RPT + k * CH, CH)])


def _tc_scale_body(xp_ref, deg_ref, xs_ref, dinv_ref):
    d = deg_ref[0, :, 0:1] + deg_ref[1, :, 0:1] + 1.0
    dinv = lax.rsqrt(d)
    xs_ref[...] = xp_ref[...] * dinv
    dinv_ref[...] = dinv


def _tc_layer_body(p_ref, xs_ref, dinv_ref, w_ref, b_ref, o_ref):
    agg = jnp.concatenate([p_ref[0], p_ref[1]], axis=-1) + xs_ref[...]
    a = agg * dinv_ref[...]
    h = jnp.dot(a, w_ref[...], preferred_element_type=jnp.float32) + b_ref[...]
    o_ref[...] = jnp.maximum(h, 0.0) * dinv_ref[...]


def _tc_out_body(p_ref, xs_ref, dinv_ref, w_ref, b_ref, wr_ref, br_ref, o_ref):
    agg = jnp.concatenate([p_ref[0], p_ref[1]], axis=-1) + xs_ref[...]
    a = agg * dinv_ref[...]
    h = jnp.dot(a, w_ref[...], preferred_element_type=jnp.float32) + b_ref[...]
    h = jnp.maximum(h, 0.0)
    o_ref[...] = jnp.dot(h, wr_ref[...], preferred_element_type=jnp.float32) + br_ref[...]


def _row_spec(w):
    return pl.BlockSpec((RPT, w), lambda i: (i, 0))


def _rep_spec(shape):
    nd = len(shape)
    return pl.BlockSpec(shape, lambda i: (0,) * nd)


_INVP = np.array(
    [32 * (t // 32)
     + ((t % 32) // 2 if t % 2 == 0 else 16 + (t % 32 - 1) // 2)
     for t in range(D)], dtype=np.int32)


def _split_cols(t):
    b = t[:, _INVP].reshape(NPAD, NC, DH).transpose(1, 0, 2).astype(jnp.bfloat16)
    return lax.bitcast_convert_type(b.reshape(NC, NPAD, DH // 2, 2), jnp.int32)


def kernel(x, edge_index, W1, b1, W2, b2, Wr, br):
    ei = edge_index.astype(jnp.int32)
    pad = jnp.full((2, EPAD - E), N, jnp.int32)
    eidx = jnp.concatenate([ei, pad], axis=1)
    eidx = eidx.reshape(2, NS, NCH, CH).transpose(1, 2, 0, 3)
    xp = jnp.pad(x, ((0, NPAD - N), (0, 0)))
    vals = jnp.concatenate([jnp.ones((CH, DEGW), jnp.float32),
                            jnp.zeros((CH, DEGW), jnp.float32)], axis=0)
    b1r = b1.reshape(1, D)
    b2r = b2.reshape(1, D)
    brr = br.reshape(1, 1)

    deg_parts = _deg_kernel(eidx, vals)

    xs, dinv = pl.pallas_call(
        _tc_scale_body,
        grid=(NS,),
        in_specs=[_row_spec(D),
                  pl.BlockSpec((NC, RPT, DEGW), lambda i: (0, i, 0))],
        out_specs=[_row_spec(D), _row_spec(1)],
        out_shape=[jax.ShapeDtypeStruct((NPAD, D), jnp.float32),
                   jax.ShapeDtypeStruct((NPAD, 1), jnp.float32)],
    )(xp, deg_parts)

    parts1 = _agg_kernel(_split_cols(xs), eidx)

    xs2 = pl.pallas_call(
        _tc_layer_body,
        grid=(NS,),
        in_specs=[pl.BlockSpec((NC, RPT, DH), lambda i: (0, i, 0)),
                  _row_spec(D), _row_spec(1),
                  _rep_spec((D, D)), _rep_spec((1, D))],
        out_specs=_row_spec(D),
        out_shape=jax.ShapeDtypeStruct((NPAD, D), jnp.float32),
    )(parts1, xs, dinv, W1, b1r)

    parts2 = _agg_kernel(_split_cols(xs2), eidx)

    out_pad = pl.pallas_call(
        _tc_out_body,
        grid=(NS,),
        in_specs=[pl.BlockSpec((NC, RPT, DH), lambda i: (0, i, 0)),
                  _row_spec(D), _row_spec(1),
                  _rep_spec((D, D)), _rep_spec((1, D)),
                  _rep_spec((D, 1)), _rep_spec((1, 1))],
        out_specs=_row_spec(1),
        out_shape=jax.ShapeDtypeStruct((NPAD, 1), jnp.float32),
    )(parts2, xs2, dinv, W2, b2r, Wr, brr)

    return out_pad[:N]


# speedup vs baseline: 1.1028x; 1.1028x over previous
import functools

import numpy as np

import jax
import jax.numpy as jnp
from jax import lax
from jax.experimental import pallas as pl
from jax.experimental.pallas import tpu as pltpu
from jax.experimental.pallas import tpu_sc as plsc

N = 10000
E = 320000
D = 128
DH = D // 2
NC = 2
NS = 16
CH = 64
SB = 6
NSUP = 54
NCH = SB * NSUP
EPAD = NS * NCH * CH
NPAD = 10240
RPT = NPAD // NS
DEGW = 8
DNCH = NCH // NC

_mesh = plsc.VectorSubcoreMesh(core_axis_name="c", subcore_axis_name="s")


@functools.partial(
    pl.kernel,
    out_type=jax.ShapeDtypeStruct((NC, NPAD, DEGW), jnp.float32),
    mesh=_mesh,
    scratch_types=[
        pltpu.VMEM((2, CH), jnp.int32),
        pltpu.VMEM((2 * CH, DEGW), jnp.float32),
        pltpu.SemaphoreType.DMA,
        pltpu.SemaphoreType.DMA,
        pltpu.VMEM_SHARED((NPAD, DEGW), jnp.float32),
    ],
)
def _deg_kernel(eidx_hbm, vals_hbm, out_hbm, dbuf, vbuf, sem_a, sem_b, acc):
    c = lax.axis_index("c")
    s = lax.axis_index("s")
    pltpu.sync_copy(vals_hbm, vbuf)
    for k in range(RPT // CH):
        pltpu.sync_copy(vbuf.at[pl.ds(CH, CH)],
                        acc.at[pl.ds(s * RPT + k * CH, CH)])
    plsc.subcore_barrier()

    lo = c * DNCH
    hi = lo + DNCH - 1
    pltpu.make_async_copy(eidx_hbm.at[s, lo, 1], dbuf.at[0], sem_a).start()
    pltpu.make_async_copy(eidx_hbm.at[s, lo + 1, 1], dbuf.at[1], sem_b).start()

    def pair(i, carry):
        j0 = lo + 2 * i
        j1 = j0 + 1
        pltpu.make_async_copy(eidx_hbm.at[s, j0, 1], dbuf.at[0], sem_a).wait()
        pltpu.sync_copy(vbuf.at[pl.ds(0, CH)], acc.at[dbuf.at[0]], add=True)
        pltpu.make_async_copy(eidx_hbm.at[s, jnp.minimum(j0 + 2, hi), 1],
                              dbuf.at[0], sem_a).start()
        pltpu.make_async_copy(eidx_hbm.at[s, j1, 1], dbuf.at[1], sem_b).wait()
        pltpu.sync_copy(vbuf.at[pl.ds(0, CH)], acc.at[dbuf.at[1]], add=True)
        pltpu.make_async_copy(eidx_hbm.at[s, jnp.minimum(j1 + 2, hi), 1],
                              dbuf.at[1], sem_b).start()
        return carry

    lax.fori_loop(0, DNCH // 2, pair, 0)
    pltpu.make_async_copy(eidx_hbm.at[s, hi, 1], dbuf.at[0], sem_a).wait()
    pltpu.make_async_copy(eidx_hbm.at[s, hi, 1], dbuf.at[1], sem_b).wait()
    plsc.subcore_barrier()

    for k in range(RPT // CH):
        stage = vbuf.at[pl.ds(0, CH)]
        pltpu.sync_copy(acc.at[pl.ds(s * RPT + k * CH, CH)], stage)
        pltpu.sync_copy(stage, out_hbm.at[c, pl.ds(s * RPT + k * CH, CH)])


def _make_agg_kernel(spmem_table):
  scratch = [
      pltpu.VMEM((2, SB, 2, CH), jnp.int32),
      pltpu.VMEM((3, CH, DH // 2), jnp.int32),
      pltpu.VMEM((2, CH, DH), jnp.float32),
      pltpu.SemaphoreType.DMA,
      pltpu.SemaphoreType.DMA,
      pltpu.SemaphoreType.DMA,
      pltpu.SemaphoreType.DMA,
      pltpu.SemaphoreType.DMA,
      pltpu.SemaphoreType.DMA,
      pltpu.VMEM_SHARED((NPAD, DH), jnp.float32),
  ]
  if spmem_table:
      scratch.append(pltpu.VMEM_SHARED((NPAD, DH // 2), jnp.int32))

  @functools.partial(
      pl.kernel,
      out_type=jax.ShapeDtypeStruct((NC, NPAD, DH), jnp.float32),
      mesh=_mesh,
      scratch_types=scratch,
      compiler_params=pltpu.CompilerParams(use_tc_tiling_on_sc=False),
  )
  def _agg(xs_hbm, eidx_hbm, out_hbm, sbuf, rows, rows_s, sem_i,
           g0, g1, g2, s0, s1, acc, *opt_tbl):
      c = lax.axis_index("c")
      s = lax.axis_index("s")
      xs_c = xs_hbm.at[c]
      tbl = opt_tbl[0] if spmem_table else xs_c
      gsem = (g0, g1, g2)
      ssem = (s0, s1)

      def g_start(qu, cc, b):
          pltpu.make_async_copy(tbl.at[sbuf.at[qu, cc, 0]],
                                rows.at[b], gsem[b]).start()

      def g_wait(qu, cc, b):
          pltpu.make_async_copy(tbl.at[sbuf.at[qu, cc, 0]],
                                rows.at[b], gsem[b]).wait()

      def s_start(qu, cc, b):
          pltpu.make_async_copy(rows_s.at[b], acc.at[sbuf.at[qu, cc, 1]],
                                ssem[b]).start()

      def s_wait(qu, cc, b):
          pltpu.make_async_copy(rows_s.at[b], acc.at[sbuf.at[qu, cc, 1]],
                                ssem[b]).wait()

      def convert(b3, b2):
          def conv_row(r, carry):
              for g in range(DH // 32):
                  w = rows[b3, r, pl.ds(g * 16, 16)]
                  lo = lax.bitcast_convert_type(w << 16, jnp.float32)
                  hi = lax.bitcast_convert_type(w & jnp.int32(-65536), jnp.float32)
                  rows_s[b2, r, pl.ds(g * 32, 16)] = lo
                  rows_s[b2, r, pl.ds(g * 32 + 16, 16)] = hi
              return carry

          lax.fori_loop(0, CH, conv_row, 0)

      def i_start(v, slot):
          pltpu.make_async_copy(eidx_hbm.at[s, pl.ds(v * SB, SB)],
                                sbuf.at[slot], sem_i).start()

      def i_wait(v, slot):
          pltpu.make_async_copy(eidx_hbm.at[s, pl.ds(v * SB, SB)],
                                sbuf.at[slot], sem_i).wait()

      def fill_zero(i, carry):
          for k in range(DH // 16):
              rows_s[0, i, pl.ds(k * 16, 16)] = jnp.zeros((16,), jnp.float32)
          return carry

      lax.fori_loop(0, CH, fill_zero, 0)
      for k in range(RPT // CH):
          pltpu.sync_copy(rows_s.at[0], acc.at[pl.ds(s * RPT + k * CH, CH)])
      if spmem_table:
          pltpu.sync_copy(xs_c.at[pl.ds(s * RPT, RPT)],
                          opt_tbl[0].at[pl.ds(s * RPT, RPT)])
      plsc.subcore_barrier()

      pltpu.sync_copy(eidx_hbm.at[s, pl.ds(0, SB)], sbuf.at[0])
      g_start(0, 0, 0)
      g_start(0, 1, 1)
      for kk in range(SB):
          b3 = kk % 3
          b2 = kk % 2
          g_wait(0, kk, b3)
          convert(b3, b2)
          s_start(0, kk, b2)
          if kk == 0:
              i_start(1, 1)
          else:
              s_wait(0, kk - 1, (kk - 1) % 2)
          if kk < SB - 2:
              g_start(0, kk + 2, (kk + 2) % 3)
          else:
              if kk == SB - 2:
                  i_wait(1, 1)
              g_start(1, kk + 2 - SB, (kk + 2) % 3)

      def super_body(v, carry):
          pv = lax.rem(v, 2)
          qv = 1 - pv
          not_last = v < NSUP - 1
          for kk in range(SB):
              b3 = kk % 3
              b2 = kk % 2
              g_wait(pv, kk, b3)
              convert(b3, b2)
              s_start(pv, kk, b2)
              if kk == 0:
                  s_wait(qv, SB - 1, 1)

                  @pl.when(v + 1 < NSUP)
                  def _():
                      i_start(v + 1, qv)
              else:
                  s_wait(pv, kk - 1, (kk - 1) % 2)
              if kk < SB - 2:
                  g_start(pv, kk + 2, (kk + 2) % 3)
              else:
                  if kk == SB - 2:
                      @pl.when(not_last)
                      def _():
                          i_wait(v + 1, qv)
                  qu = jnp.where(not_last, qv, pv)
                  cc = jnp.where(not_last, kk + 2 - SB, SB - 1)
                  g_start(qu, cc, (kk + 2) % 3)
          return carry

      lax.fori_loop(1, NSUP, super_body, 0)

      pu_last = (NSUP - 1) % 2
      s_wait(pu_last, SB - 1, (SB - 1) % 2)
      g_wait(pu_last, SB - 1, 0)
      g_wait(pu_last, SB - 1, 1)
      plsc.subcore_barrier()

      for k in range(RPT // CH):
          buf = rows_s.at[k % 2]
          pltpu.sync_copy(acc.at[pl.ds(s * RPT + k * CH, CH)], buf)
          pltpu.sync_copy(buf, out_hbm.at[c, pl.ds(s * RPT + k * CH, CH)])


  return _agg


def _tc_scale_body(xp_ref, deg_ref, xs_ref, dinv_ref):
    d = deg_ref[0, :, 0:1] + deg_ref[1, :, 0:1] + 1.0
    dinv = lax.rsqrt(d)
    xs_ref[...] = xp_ref[...] * dinv
    dinv_ref[...] = dinv


def _tc_layer_body(p_ref, xs_ref, dinv_ref, w_ref, b_ref, o_ref):
    agg = jnp.concatenate([p_ref[0], p_ref[1]], axis=-1) + xs_ref[...]
    a = agg * dinv_ref[...]
    h = jnp.dot(a, w_ref[...], preferred_element_type=jnp.float32) + b_ref[...]
    o_ref[...] = jnp.maximum(h, 0.0) * dinv_ref[...]


def _tc_out_body(p_ref, xs_ref, dinv_ref, w_ref, b_ref, wr_ref, br_ref, o_ref):
    agg = jnp.concatenate([p_ref[0], p_ref[1]], axis=-1) + xs_ref[...]
    a = agg * dinv_ref[...]
    h = jnp.dot(a, w_ref[...], preferred_element_type=jnp.float32) + b_ref[...]
    h = jnp.maximum(h, 0.0)
    o_ref[...] = jnp.dot(h, wr_ref[...], preferred_element_type=jnp.float32) + br_ref[...]


def _row_spec(w):
    return pl.BlockSpec((RPT, w), lambda i: (i, 0))


def _rep_spec(shape):
    nd = len(shape)
    return pl.BlockSpec(shape, lambda i: (0,) * nd)


_INVP = np.array(
    [32 * (t // 32)
     + ((t % 32) // 2 if t % 2 == 0 else 16 + (t % 32 - 1) // 2)
     for t in range(D)], dtype=np.int32)


def _split_cols(t):
    b = t[:, _INVP].reshape(NPAD, NC, DH).transpose(1, 0, 2).astype(jnp.bfloat16)
    return lax.bitcast_convert_type(b.reshape(NC, NPAD, DH // 2, 2), jnp.int32)


_agg_kernel = _make_agg_kernel(False)
_agg_kernel_sp = _make_agg_kernel(True)


def kernel(x, edge_index, W1, b1, W2, b2, Wr, br):
    ei = edge_index.astype(jnp.int32)
    pad = jnp.full((2, EPAD - E), N, jnp.int32)
    eidx = jnp.concatenate([ei, pad], axis=1)
    eidx = eidx.reshape(2, NS, NCH, CH).transpose(1, 2, 0, 3)
    xp = jnp.pad(x, ((0, NPAD - N), (0, 0)))
    vals = jnp.concatenate([jnp.ones((CH, DEGW), jnp.float32),
                            jnp.zeros((CH, DEGW), jnp.float32)], axis=0)
    b1r = b1.reshape(1, D)
    b2r = b2.reshape(1, D)
    brr = br.reshape(1, 1)

    deg_parts = _deg_kernel(eidx, vals)

    xs, dinv = pl.pallas_call(
        _tc_scale_body,
        grid=(NS,),
        in_specs=[_row_spec(D),
                  pl.BlockSpec((NC, RPT, DEGW), lambda i: (0, i, 0))],
        out_specs=[_row_spec(D), _row_spec(1)],
        out_shape=[jax.ShapeDtypeStruct((NPAD, D), jnp.float32),
                   jax.ShapeDtypeStruct((NPAD, 1), jnp.float32)],
    )(xp, deg_parts)

    parts1 = _agg_kernel_sp(_split_cols(xs), eidx)

    xs2 = pl.pallas_call(
        _tc_layer_body,
        grid=(NS,),
        in_specs=[pl.BlockSpec((NC, RPT, DH), lambda i: (0, i, 0)),
                  _row_spec(D), _row_spec(1),
                  _rep_spec((D, D)), _rep_spec((1, D))],
        out_specs=_row_spec(D),
        out_shape=jax.ShapeDtypeStruct((NPAD, D), jnp.float32),
    )(parts1, xs, dinv, W1, b1r)

    parts2 = _agg_kernel(_split_cols(xs2), eidx)

    out_pad = pl.pallas_call(
        _tc_out_body,
        grid=(NS,),
        in_specs=[pl.BlockSpec((NC, RPT, DH), lambda i: (0, i, 0)),
                  _row_spec(D), _row_spec(1),
                  _rep_spec((D, D)), _rep_spec((1, D)),
                  _rep_spec((D, 1)), _rep_spec((1, 1))],
        out_specs=_row_spec(1),
        out_shape=jax.ShapeDtypeStruct((NPAD, 1), jnp.float32),
    )(parts2, xs2, dinv, W2, b2r, Wr, brr)

    return out_pad[:N]


# speedup vs baseline: 1.1498x; 1.0427x over previous
import functools

import numpy as np

import jax
import jax.numpy as jnp
from jax import lax
from jax.experimental import pallas as pl
from jax.experimental.pallas import tpu as pltpu
from jax.experimental.pallas import tpu_sc as plsc

N = 10000
E = 320000
D = 128
DH = D // 2
NC = 2
NS = 16
CH = 64
SB = 6
NSUP = 54
NCH = SB * NSUP
EPAD = NS * NCH * CH
NPAD = 10240
RPT = NPAD // NS
DEGW = 8
DNCH = NCH // NC
SBD = 6
NSUPD = DNCH // SBD

_mesh = plsc.VectorSubcoreMesh(core_axis_name="c", subcore_axis_name="s")


@functools.partial(
    pl.kernel,
    out_type=jax.ShapeDtypeStruct((NC, NPAD, DEGW), jnp.float32),
    mesh=_mesh,
    scratch_types=[
        pltpu.VMEM((2, SBD, CH), jnp.int32),
        pltpu.VMEM((2 * CH, DEGW), jnp.float32),
        pltpu.SemaphoreType.DMA,
        pltpu.SemaphoreType.DMA,
        pltpu.VMEM_SHARED((NPAD, DEGW), jnp.float32),
    ],
)
def _deg_kernel(eidx_hbm, vals_hbm, out_hbm, dbuf, vbuf, sem_a, sem_b, acc):
    c = lax.axis_index("c")
    s = lax.axis_index("s")
    pltpu.sync_copy(vals_hbm, vbuf)
    for k in range(RPT // CH):
        pltpu.sync_copy(vbuf.at[pl.ds(CH, CH)],
                        acc.at[pl.ds(s * RPT + k * CH, CH)])
    plsc.subcore_barrier()

    lo = c * DNCH

    def i_start(u, slot):
        pltpu.make_async_copy(eidx_hbm.at[s, pl.ds(lo + u * SBD, SBD), 1],
                              dbuf.at[slot], sem_a).start()

    def i_wait(u, slot):
        pltpu.make_async_copy(eidx_hbm.at[s, pl.ds(lo + u * SBD, SBD), 1],
                              dbuf.at[slot], sem_a).wait()

    def sc_start(slot, k):
        pltpu.make_async_copy(vbuf.at[pl.ds(0, CH)],
                              acc.at[dbuf.at[slot, k]], sem_b).start()

    def sc_wait(slot, k):
        pltpu.make_async_copy(vbuf.at[pl.ds(0, CH)],
                              acc.at[dbuf.at[slot, k]], sem_b).wait()

    pltpu.sync_copy(eidx_hbm.at[s, pl.ds(lo, SBD), 1], dbuf.at[0])
    i_start(1, 1)
    for k in range(SBD):
        sc_start(0, k)

    def super_body(u, carry):
        pu = lax.rem(u, 2)
        qu = 1 - pu
        i_wait(u, pu)
        for k in range(SBD):
            sc_wait(qu, k)

        @pl.when(u + 1 < NSUPD)
        def _():
            i_start(u + 1, qu)

        for k in range(SBD):
            sc_start(pu, k)
        return carry

    lax.fori_loop(1, NSUPD, super_body, 0)
    pl_last = (NSUPD - 1) % 2
    for k in range(SBD):
        sc_wait(pl_last, k)
    plsc.subcore_barrier()

    for k in range(RPT // CH):
        stage = vbuf.at[pl.ds(0, CH)]
        pltpu.sync_copy(acc.at[pl.ds(s * RPT + k * CH, CH)], stage)
        pltpu.sync_copy(stage, out_hbm.at[c, pl.ds(s * RPT + k * CH, CH)])


def _make_agg_kernel(spmem_table):
  scratch = [
      pltpu.VMEM((2, SB, 2, CH), jnp.int32),
      pltpu.VMEM((3, CH, DH // 2), jnp.int32),
      pltpu.VMEM((2, CH, DH), jnp.float32),
      pltpu.SemaphoreType.DMA,
      pltpu.SemaphoreType.DMA,
      pltpu.SemaphoreType.DMA,
      pltpu.SemaphoreType.DMA,
      pltpu.SemaphoreType.DMA,
      pltpu.SemaphoreType.DMA,
      pltpu.VMEM_SHARED((NPAD, DH), jnp.float32),
  ]
  if spmem_table:
      scratch.append(pltpu.VMEM_SHARED((NPAD, DH // 2), jnp.int32))

  @functools.partial(
      pl.kernel,
      out_type=jax.ShapeDtypeStruct((NC, NPAD, DH), jnp.float32),
      mesh=_mesh,
      scratch_types=scratch,
      compiler_params=pltpu.CompilerParams(use_tc_tiling_on_sc=False),
  )
  def _agg(xs_hbm, eidx_hbm, out_hbm, sbuf, rows, rows_s, sem_i,
           g0, g1, g2, s0, s1, acc, *opt_tbl):
      c = lax.axis_index("c")
      s = lax.axis_index("s")
      xs_c = xs_hbm.at[c]
      tbl = opt_tbl[0] if spmem_table else xs_c
      gsem = (g0, g1, g2)
      ssem = (s0, s1)

      def g_start(qu, cc, b):
          pltpu.make_async_copy(tbl.at[sbuf.at[qu, cc, 0]],
                                rows.at[b], gsem[b]).start()

      def g_wait(qu, cc, b):
          pltpu.make_async_copy(tbl.at[sbuf.at[qu, cc, 0]],
                                rows.at[b], gsem[b]).wait()

      def s_start(qu, cc, b):
          pltpu.make_async_copy(rows_s.at[b], acc.at[sbuf.at[qu, cc, 1]],
                                ssem[b]).start()

      def s_wait(qu, cc, b):
          pltpu.make_async_copy(rows_s.at[b], acc.at[sbuf.at[qu, cc, 1]],
                                ssem[b]).wait()

      def convert(b3, b2):
          def conv_row(r, carry):
              for g in range(DH // 32):
                  w = rows[b3, r, pl.ds(g * 16, 16)]
                  lo = lax.bitcast_convert_type(w << 16, jnp.float32)
                  hi = lax.bitcast_convert_type(w & jnp.int32(-65536), jnp.float32)
                  rows_s[b2, r, pl.ds(g * 32, 16)] = lo
                  rows_s[b2, r, pl.ds(g * 32 + 16, 16)] = hi
              return carry

          lax.fori_loop(0, CH, conv_row, 0)

      def i_start(v, slot):
          pltpu.make_async_copy(eidx_hbm.at[s, pl.ds(v * SB, SB)],
                                sbuf.at[slot], sem_i).start()

      def i_wait(v, slot):
          pltpu.make_async_copy(eidx_hbm.at[s, pl.ds(v * SB, SB)],
                                sbuf.at[slot], sem_i).wait()

      def fill_zero(i, carry):
          for k in range(DH // 16):
              rows_s[0, i, pl.ds(k * 16, 16)] = jnp.zeros((16,), jnp.float32)
          return carry

      lax.fori_loop(0, CH, fill_zero, 0)
      for k in range(RPT // CH):
          pltpu.sync_copy(rows_s.at[0], acc.at[pl.ds(s * RPT + k * CH, CH)])
      if spmem_table:
          pltpu.sync_copy(xs_c.at[pl.ds(s * RPT, RPT)],
                          opt_tbl[0].at[pl.ds(s * RPT, RPT)])
      plsc.subcore_barrier()

      pltpu.sync_copy(eidx_hbm.at[s, pl.ds(0, SB)], sbuf.at[0])
      g_start(0, 0, 0)
      g_start(0, 1, 1)
      for kk in range(SB):
          b3 = kk % 3
          b2 = kk % 2
          g_wait(0, kk, b3)
          convert(b3, b2)
          s_start(0, kk, b2)
          if kk == 0:
              i_start(1, 1)
          else:
              s_wait(0, kk - 1, (kk - 1) % 2)
          if kk < SB - 2:
              g_start(0, kk + 2, (kk + 2) % 3)
          else:
              if kk == SB - 2:
                  i_wait(1, 1)
              g_start(1, kk + 2 - SB, (kk + 2) % 3)

      def super_body(v, carry):
          pv = lax.rem(v, 2)
          qv = 1 - pv
          not_last = v < NSUP - 1
          for kk in range(SB):
              b3 = kk % 3
              b2 = kk % 2
              g_wait(pv, kk, b3)
              convert(b3, b2)
              s_start(pv, kk, b2)
              if kk == 0:
                  s_wait(qv, SB - 1, 1)

                  @pl.when(v + 1 < NSUP)
                  def _():
                      i_start(v + 1, qv)
              else:
                  s_wait(pv, kk - 1, (kk - 1) % 2)
              if kk < SB - 2:
                  g_start(pv, kk + 2, (kk + 2) % 3)
              else:
                  if kk == SB - 2:
                      @pl.when(not_last)
                      def _():
                          i_wait(v + 1, qv)
                  qu = jnp.where(not_last, qv, pv)
                  cc = jnp.where(not_last, kk + 2 - SB, SB - 1)
                  g_start(qu, cc, (kk + 2) % 3)
          return carry

      lax.fori_loop(1, NSUP, super_body, 0)

      pu_last = (NSUP - 1) % 2
      s_wait(pu_last, SB - 1, (SB - 1) % 2)
      g_wait(pu_last, SB - 1, 0)
      g_wait(pu_last, SB - 1, 1)
      plsc.subcore_barrier()

      for k in range(RPT // CH):
          buf = rows_s.at[k % 2]
          pltpu.sync_copy(acc.at[pl.ds(s * RPT + k * CH, CH)], buf)
          pltpu.sync_copy(buf, out_hbm.at[c, pl.ds(s * RPT + k * CH, CH)])


  return _agg


def _tc_scale_body(xp_ref, deg_ref, xs_ref, dinv_ref):
    d = deg_ref[0, :, 0:1] + deg_ref[1, :, 0:1] + 1.0
    dinv = lax.rsqrt(d)
    xs_ref[...] = xp_ref[...] * dinv
    dinv_ref[...] = dinv


def _tc_layer_body(p_ref, xs_ref, dinv_ref, w_ref, b_ref, o_ref):
    agg = jnp.concatenate([p_ref[0], p_ref[1]], axis=-1) + xs_ref[...]
    a = agg * dinv_ref[...]
    h = jnp.dot(a, w_ref[...], preferred_element_type=jnp.float32) + b_ref[...]
    o_ref[...] = jnp.maximum(h, 0.0) * dinv_ref[...]


def _tc_out_body(p_ref, xs_ref, dinv_ref, w_ref, b_ref, wr_ref, br_ref, o_ref):
    agg = jnp.concatenate([p_ref[0], p_ref[1]], axis=-1) + xs_ref[...]
    a = agg * dinv_ref[...]
    h = jnp.dot(a, w_ref[...], preferred_element_type=jnp.float32) + b_ref[...]
    h = jnp.maximum(h, 0.0)
    o_ref[...] = jnp.dot(h, wr_ref[...], preferred_element_type=jnp.float32) + br_ref[...]


def _row_spec(w):
    return pl.BlockSpec((RPT, w), lambda i: (i, 0))


def _rep_spec(shape):
    nd = len(shape)
    return pl.BlockSpec(shape, lambda i: (0,) * nd)


_INVP = np.array(
    [32 * (t // 32)
     + ((t % 32) // 2 if t % 2 == 0 else 16 + (t % 32 - 1) // 2)
     for t in range(D)], dtype=np.int32)


def _split_cols(t):
    b = t[:, _INVP].reshape(NPAD, NC, DH).transpose(1, 0, 2).astype(jnp.bfloat16)
    return lax.bitcast_convert_type(b.reshape(NC, NPAD, DH // 2, 2), jnp.int32)


_agg_kernel = _make_agg_kernel(False)
_agg_kernel_sp = _make_agg_kernel(True)


def kernel(x, edge_index, W1, b1, W2, b2, Wr, br):
    ei = edge_index.astype(jnp.int32)
    pad = jnp.full((2, EPAD - E), N, jnp.int32)
    eidx = jnp.concatenate([ei, pad], axis=1)
    eidx = eidx.reshape(2, NS, NCH, CH).transpose(1, 2, 0, 3)
    xp = jnp.pad(x, ((0, NPAD - N), (0, 0)))
    vals = jnp.concatenate([jnp.ones((CH, DEGW), jnp.float32),
                            jnp.zeros((CH, DEGW), jnp.float32)], axis=0)
    b1r = b1.reshape(1, D)
    b2r = b2.reshape(1, D)
    brr = br.reshape(1, 1)

    deg_parts = _deg_kernel(eidx, vals)

    xs, dinv = pl.pallas_call(
        _tc_scale_body,
        grid=(NS,),
        in_specs=[_row_spec(D),
                  pl.BlockSpec((NC, RPT, DEGW), lambda i: (0, i, 0))],
        out_specs=[_row_spec(D), _row_spec(1)],
        out_shape=[jax.ShapeDtypeStruct((NPAD, D), jnp.float32),
                   jax.ShapeDtypeStruct((NPAD, 1), jnp.float32)],
    )(xp, deg_parts)

    parts1 = _agg_kernel_sp(_split_cols(xs), eidx)

    xs2 = pl.pallas_call(
        _tc_layer_body,
        grid=(NS,),
        in_specs=[pl.BlockSpec((NC, RPT, DH), lambda i: (0, i, 0)),
                  _row_spec(D), _row_spec(1),
                  _rep_spec((D, D)), _rep_spec((1, D))],
        out_specs=_row_spec(D),
        out_shape=jax.ShapeDtypeStruct((NPAD, D), jnp.float32),
    )(parts1, xs, dinv, W1, b1r)

    parts2 = _agg_kernel(_split_cols(xs2), eidx)

    out_pad = pl.pallas_call(
        _tc_out_body,
        grid=(NS,),
        in_specs=[pl.BlockSpec((NC, RPT, DH), lambda i: (0, i, 0)),
                  _row_spec(D), _row_spec(1),
                  _rep_spec((D, D)), _rep_spec((1, D)),
                  _rep_spec((D, 1)), _rep_spec((1, 1))],
        out_specs=_row_spec(1),
        out_shape=jax.ShapeDtypeStruct((NPAD, 1), jnp.float32),
    )(parts2, xs2, dinv, W2, b2r, Wr, brr)

    return out_pad[:N]


# speedup vs baseline: 1.2275x; 1.0675x over previous
import functools

import numpy as np

import jax
import jax.numpy as jnp
from jax import lax
from jax.experimental import pallas as pl
from jax.experimental.pallas import tpu as pltpu
from jax.experimental.pallas import tpu_sc as plsc

N = 10000
E = 320000
D = 128
DH = D // 2
NC = 2
NS = 16
CH = 64
SB = 10
NSUP = 32
NCH = SB * NSUP
EPAD = NS * NCH * CH
NPAD = 10240
RPT = NPAD // NS
DEGW = 8
DNCH = NCH // NC
SBD = 8
NSUPD = DNCH // SBD

_mesh = plsc.VectorSubcoreMesh(core_axis_name="c", subcore_axis_name="s")


@functools.partial(
    pl.kernel,
    out_type=jax.ShapeDtypeStruct((NC, NPAD, DEGW), jnp.float32),
    mesh=_mesh,
    scratch_types=[
        pltpu.VMEM((2, SBD, CH), jnp.int32),
        pltpu.VMEM((2 * CH, DEGW), jnp.float32),
        pltpu.SemaphoreType.DMA,
        pltpu.SemaphoreType.DMA,
        pltpu.VMEM_SHARED((NPAD, DEGW), jnp.float32),
    ],
)
def _deg_kernel(eidx_hbm, vals_hbm, out_hbm, dbuf, vbuf, sem_a, sem_b, acc):
    c = lax.axis_index("c")
    s = lax.axis_index("s")
    pltpu.sync_copy(vals_hbm, vbuf)
    for k in range(RPT // CH):
        pltpu.sync_copy(vbuf.at[pl.ds(CH, CH)],
                        acc.at[pl.ds(s * RPT + k * CH, CH)])
    plsc.subcore_barrier()

    lo = c * DNCH

    def i_start(u, slot):
        pltpu.make_async_copy(eidx_hbm.at[s, pl.ds(lo + u * SBD, SBD), 1],
                              dbuf.at[slot], sem_a).start()

    def i_wait(u, slot):
        pltpu.make_async_copy(eidx_hbm.at[s, pl.ds(lo + u * SBD, SBD), 1],
                              dbuf.at[slot], sem_a).wait()

    def sc_start(slot, k):
        pltpu.make_async_copy(vbuf.at[pl.ds(0, CH)],
                              acc.at[dbuf.at[slot, k]], sem_b).start()

    def sc_wait(slot, k):
        pltpu.make_async_copy(vbuf.at[pl.ds(0, CH)],
                              acc.at[dbuf.at[slot, k]], sem_b).wait()

    pltpu.sync_copy(eidx_hbm.at[s, pl.ds(lo, SBD), 1], dbuf.at[0])
    i_start(1, 1)
    for k in range(SBD):
        sc_start(0, k)

    def super_body(u, carry):
        pu = lax.rem(u, 2)
        qu = 1 - pu
        i_wait(u, pu)
        for k in range(SBD):
            sc_wait(qu, k)

        @pl.when(u + 1 < NSUPD)
        def _():
            i_start(u + 1, qu)

        for k in range(SBD):
            sc_start(pu, k)
        return carry

    lax.fori_loop(1, NSUPD, super_body, 0)
    pl_last = (NSUPD - 1) % 2
    for k in range(SBD):
        sc_wait(pl_last, k)
    plsc.subcore_barrier()

    for k in range(RPT // CH):
        stage = vbuf.at[pl.ds(0, CH)]
        pltpu.sync_copy(acc.at[pl.ds(s * RPT + k * CH, CH)], stage)
        pltpu.sync_copy(stage, out_hbm.at[c, pl.ds(s * RPT + k * CH, CH)])


def _make_agg_kernel(spmem_table):
  scratch = [
      pltpu.VMEM((2, SB, 2, CH), jnp.int32),
      pltpu.VMEM((5, CH, DH // 2), jnp.int32),
      pltpu.VMEM((2, CH, DH), jnp.float32),
      pltpu.SemaphoreType.DMA,
      pltpu.SemaphoreType.DMA,
      pltpu.SemaphoreType.DMA,
      pltpu.SemaphoreType.DMA,
      pltpu.SemaphoreType.DMA,
      pltpu.SemaphoreType.DMA,
      pltpu.SemaphoreType.DMA,
      pltpu.SemaphoreType.DMA,
      pltpu.VMEM_SHARED((NPAD, DH), jnp.float32),
  ]
  if spmem_table:
      scratch.append(pltpu.VMEM_SHARED((NPAD, DH // 2), jnp.int32))

  @functools.partial(
      pl.kernel,
      out_type=jax.ShapeDtypeStruct((NC, NPAD, DH), jnp.float32),
      mesh=_mesh,
      scratch_types=scratch,
      compiler_params=pltpu.CompilerParams(use_tc_tiling_on_sc=False),
  )
  def _agg(xs_hbm, eidx_hbm, out_hbm, sbuf, rows, rows_s, sem_i,
           g0, g1, g2, g3, g4, s0, s1, acc, *opt_tbl):
      c = lax.axis_index("c")
      s = lax.axis_index("s")
      xs_c = xs_hbm.at[c]
      tbl = opt_tbl[0] if spmem_table else xs_c
      gsem = (g0, g1, g2, g3, g4)
      ssem = (s0, s1)

      def g_start(qu, cc, b):
          pltpu.make_async_copy(tbl.at[sbuf.at[qu, cc, 0]],
                                rows.at[b], gsem[b]).start()

      def g_wait(qu, cc, b):
          pltpu.make_async_copy(tbl.at[sbuf.at[qu, cc, 0]],
                                rows.at[b], gsem[b]).wait()

      def s_start(qu, cc, b):
          pltpu.make_async_copy(rows_s.at[b], acc.at[sbuf.at[qu, cc, 1]],
                                ssem[b]).start()

      def s_wait(qu, cc, b):
          pltpu.make_async_copy(rows_s.at[b], acc.at[sbuf.at[qu, cc, 1]],
                                ssem[b]).wait()

      def convert(b3, b2):
          def conv_row(r, carry):
              for g in range(DH // 32):
                  w = rows[b3, r, pl.ds(g * 16, 16)]
                  lo = lax.bitcast_convert_type(w << 16, jnp.float32)
                  hi = lax.bitcast_convert_type(w & jnp.int32(-65536), jnp.float32)
                  rows_s[b2, r, pl.ds(g * 32, 16)] = lo
                  rows_s[b2, r, pl.ds(g * 32 + 16, 16)] = hi
              return carry

          lax.fori_loop(0, CH, conv_row, 0)

      def i_start(v, slot):
          pltpu.make_async_copy(eidx_hbm.at[s, pl.ds(v * SB, SB)],
                                sbuf.at[slot], sem_i).start()

      def i_wait(v, slot):
          pltpu.make_async_copy(eidx_hbm.at[s, pl.ds(v * SB, SB)],
                                sbuf.at[slot], sem_i).wait()

      def fill_zero(i, carry):
          for k in range(DH // 16):
              rows_s[0, i, pl.ds(k * 16, 16)] = jnp.zeros((16,), jnp.float32)
          return carry

      lax.fori_loop(0, CH, fill_zero, 0)
      for k in range(RPT // CH):
          pltpu.sync_copy(rows_s.at[0], acc.at[pl.ds(s * RPT + k * CH, CH)])
      if spmem_table:
          pltpu.sync_copy(xs_c.at[pl.ds(s * RPT, RPT)],
                          opt_tbl[0].at[pl.ds(s * RPT, RPT)])
      plsc.subcore_barrier()

      pltpu.sync_copy(eidx_hbm.at[s, pl.ds(0, SB)], sbuf.at[0])
      for j in range(4):
          g_start(0, j, j)
      for kk in range(SB):
          b3 = kk % 5
          b2 = kk % 2
          g_wait(0, kk, b3)
          convert(b3, b2)
          s_start(0, kk, b2)
          if kk == 0:
              i_start(1, 1)
          else:
              s_wait(0, kk - 1, (kk - 1) % 2)
          if kk < SB - 4:
              g_start(0, kk + 4, (kk + 4) % 5)
          else:
              if kk == SB - 4:
                  i_wait(1, 1)
              g_start(1, kk + 4 - SB, (kk + 4) % 5)

      def super_body(v, carry):
          pv = lax.rem(v, 2)
          qv = 1 - pv
          not_last = v < NSUP - 1
          for kk in range(SB):
              b3 = kk % 5
              b2 = kk % 2
              g_wait(pv, kk, b3)
              convert(b3, b2)
              s_start(pv, kk, b2)
              if kk == 0:
                  s_wait(qv, SB - 1, 1)

                  @pl.when(v + 1 < NSUP)
                  def _():
                      i_start(v + 1, qv)
              else:
                  s_wait(pv, kk - 1, (kk - 1) % 2)
              if kk < SB - 4:
                  g_start(pv, kk + 4, (kk + 4) % 5)
              else:
                  if kk == SB - 4:
                      @pl.when(not_last)
                      def _():
                          i_wait(v + 1, qv)
                  qu = jnp.where(not_last, qv, pv)
                  cc = jnp.where(not_last, kk + 4 - SB, SB - 1)
                  g_start(qu, cc, (kk + 4) % 5)
          return carry

      lax.fori_loop(1, NSUP, super_body, 0)

      pu_last = (NSUP - 1) % 2
      s_wait(pu_last, SB - 1, (SB - 1) % 2)
      for j in range(4):
          g_wait(pu_last, SB - 1, j)
      plsc.subcore_barrier()

      for k in range(RPT // CH):
          buf = rows_s.at[k % 2]
          pltpu.sync_copy(acc.at[pl.ds(s * RPT + k * CH, CH)], buf)
          pltpu.sync_copy(buf, out_hbm.at[c, pl.ds(s * RPT + k * CH, CH)])


  return _agg


def _tc_scale_body(xp_ref, deg_ref, xs_ref, dinv_ref):
    d = deg_ref[0, :, 0:1] + deg_ref[1, :, 0:1] + 1.0
    dinv = lax.rsqrt(d)
    xs_ref[...] = xp_ref[...] * dinv
    dinv_ref[...] = dinv


def _tc_layer_body(p_ref, xs_ref, dinv_ref, w_ref, b_ref, o_ref):
    agg = jnp.concatenate([p_ref[0], p_ref[1]], axis=-1) + xs_ref[...]
    a = agg * dinv_ref[...]
    h = jnp.dot(a, w_ref[...], preferred_element_type=jnp.float32) + b_ref[...]
    o_ref[...] = jnp.maximum(h, 0.0) * dinv_ref[...]


def _tc_out_body(p_ref, xs_ref, dinv_ref, w_ref, b_ref, wr_ref, br_ref, o_ref):
    agg = jnp.concatenate([p_ref[0], p_ref[1]], axis=-1) + xs_ref[...]
    a = agg * dinv_ref[...]
    h = jnp.dot(a, w_ref[...], preferred_element_type=jnp.float32) + b_ref[...]
    h = jnp.maximum(h, 0.0)
    o_ref[...] = jnp.dot(h, wr_ref[...], preferred_element_type=jnp.float32) + br_ref[...]


def _row_spec(w):
    return pl.BlockSpec((RPT, w), lambda i: (i, 0))


def _rep_spec(shape):
    nd = len(shape)
    return pl.BlockSpec(shape, lambda i: (0,) * nd)


_INVP = np.array(
    [32 * (t // 32)
     + ((t % 32) // 2 if t % 2 == 0 else 16 + (t % 32 - 1) // 2)
     for t in range(D)], dtype=np.int32)


def _split_cols(t):
    b = t[:, _INVP].reshape(NPAD, NC, DH).transpose(1, 0, 2).astype(jnp.bfloat16)
    return lax.bitcast_convert_type(b.reshape(NC, NPAD, DH // 2, 2), jnp.int32)


_agg_kernel = _make_agg_kernel(False)
_agg_kernel_sp = _make_agg_kernel(True)


def kernel(x, edge_index, W1, b1, W2, b2, Wr, br):
    ei = edge_index.astype(jnp.int32)
    pad = jnp.full((2, EPAD - E), N, jnp.int32)
    eidx = jnp.concatenate([ei, pad], axis=1)
    eidx = eidx.reshape(2, NS, NCH, CH).transpose(1, 2, 0, 3)
    xp = jnp.pad(x, ((0, NPAD - N), (0, 0)))
    vals = jnp.concatenate([jnp.ones((CH, DEGW), jnp.float32),
                            jnp.zeros((CH, DEGW), jnp.float32)], axis=0)
    b1r = b1.reshape(1, D)
    b2r = b2.reshape(1, D)
    brr = br.reshape(1, 1)

    deg_parts = _deg_kernel(eidx, vals)

    xs, dinv = pl.pallas_call(
        _tc_scale_body,
        grid=(NS,),
        in_specs=[_row_spec(D),
                  pl.BlockSpec((NC, RPT, DEGW), lambda i: (0, i, 0))],
        out_specs=[_row_spec(D), _row_spec(1)],
        out_shape=[jax.ShapeDtypeStruct((NPAD, D), jnp.float32),
                   jax.ShapeDtypeStruct((NPAD, 1), jnp.float32)],
    )(xp, deg_parts)

    parts1 = _agg_kernel_sp(_split_cols(xs), eidx)

    xs2 = pl.pallas_call(
        _tc_layer_body,
        grid=(NS,),
        in_specs=[pl.BlockSpec((NC, RPT, DH), lambda i: (0, i, 0)),
                  _row_spec(D), _row_spec(1),
                  _rep_spec((D, D)), _rep_spec((1, D))],
        out_specs=_row_spec(D),
        out_shape=jax.ShapeDtypeStruct((NPAD, D), jnp.float32),
    )(parts1, xs, dinv, W1, b1r)

    parts2 = _agg_kernel(_split_cols(xs2), eidx)

    out_pad = pl.pallas_call(
        _tc_out_body,
        grid=(NS,),
        in_specs=[pl.BlockSpec((NC, RPT, DH), lambda i: (0, i, 0)),
                  _row_spec(D), _row_spec(1),
                  _rep_spec((D, D)), _rep_spec((1, D)),
                  _rep_spec((D, 1)), _rep_spec((1, 1))],
        out_specs=_row_spec(1),
        out_shape=jax.ShapeDtypeStruct((NPAD, 1), jnp.float32),
    )(parts2, xs2, dinv, W2, b2r, Wr, brr)

    return out_pad[:N]
